# baseline (speedup 1.0000x reference)
import functools

import jax
import jax.numpy as jnp
from jax import lax
from jax.experimental import pallas as pl
from jax.experimental.pallas import tpu as pltpu
from jax.experimental.pallas import tpu_sc as plsc

_MAX_OUT = 100
_IOU_THR = 0.5
_SCORE_THR = 0.05
_PRE_NMS = 1000
_NEG = -1e9
_BIG = 2**30
_K = 1024
_NW = 32
_DEAD = 1016


def _select_body(sc_ref, s0_ref, gd_ref):
    sc = sc_ref[0]
    C, N = sc.shape
    bits = lax.bitcast_convert_type(sc, jnp.int32)
    lo = jnp.zeros((C, 1), jnp.int32)
    hi = jnp.max(bits, axis=1, keepdims=True) + 1

    def bs_step(_, carry):
        lo, hi = carry
        mid = (lo + hi) // 2
        cnt = jnp.sum((bits >= mid).astype(jnp.int32), axis=1, keepdims=True)
        ge = cnt >= _PRE_NMS
        return jnp.where(ge, mid, lo), jnp.where(ge, hi, mid)

    lo, hi = lax.fori_loop(0, 31, bs_step, (lo, hi))
    gt = bits > lo
    eq = bits == lo
    n_gt = jnp.sum(gt.astype(jnp.int32), axis=1, keepdims=True)
    m = _PRE_NMS - n_gt
    eqrank = eq.astype(jnp.int32)
    shift = 1
    while shift < N:
        z = jnp.zeros((C, shift), jnp.int32)
        eqrank = eqrank + jnp.concatenate([z, eqrank[:, :N - shift]], axis=1)
        shift *= 2
    sel = gt | (eq & (eqrank <= m))
    m2 = sel & (sc > _SCORE_THR)
    s0_ref[0] = jnp.where(m2, sc, -1.0)
    rank = m2.astype(jnp.int32)
    shift = 1
    while shift < N:
        z = jnp.zeros((C, shift), jnp.int32)
        rank = rank + jnp.concatenate([z, rank[:, :N - shift]], axis=1)
        shift *= 2
    rank = rank - 1
    cls0 = lax.broadcasted_iota(jnp.int32, (C, 1), 0)
    base = (pl.program_id(0) * C + cls0) * _K
    gd_ref[0] = jnp.where(m2, base + rank, base + _DEAD)


def _compact_sc(s0, gdest, bxt):
    R, N = s0.shape
    B = bxt.shape[0]
    ncls = R // B
    rows_per = R // _NW
    mesh = plsc.VectorSubcoreMesh(core_axis_name="c", subcore_axis_name="s")

    def body(s0_hbm, gd_hbm, bx_hbm, os_hbm, oy1_hbm, ox1_hbm, oy2_hbm,
             ox2_hbm, sv, dv, y1v, x1v, y2v, x2v, obneg, obzero):
        wid = lax.axis_index("s") * 2 + lax.axis_index("c")
        neg1 = jnp.full((16,), -1.0, jnp.float32)
        zero = jnp.zeros((16,), jnp.float32)

        def init_chunk(t, _):
            off = pl.multiple_of(t * 16, 16)
            obneg[pl.ds(off, 16)] = neg1
            obzero[pl.ds(off, 16)] = zero
            return 0

        lax.fori_loop(0, _K // 16, init_chunk, 0)

        for j in range(rows_per):
            r = wid * rows_per + j
            b = r // ncls
            pltpu.sync_copy(s0_hbm.at[pl.ds(pl.multiple_of(r * N, 8), N)], sv)
            pltpu.sync_copy(gd_hbm.at[pl.ds(pl.multiple_of(r * N, 8), N)], dv)
            bb = b * 4 * N
            pltpu.sync_copy(bx_hbm.at[pl.ds(pl.multiple_of(bb, 8), N)], y1v)
            pltpu.sync_copy(bx_hbm.at[pl.ds(pl.multiple_of(bb + N, 8), N)], x1v)
            pltpu.sync_copy(bx_hbm.at[pl.ds(pl.multiple_of(bb + 2 * N, 8), N)], y2v)
            pltpu.sync_copy(bx_hbm.at[pl.ds(pl.multiple_of(bb + 3 * N, 8), N)], x2v)

            ro = pl.ds(pl.multiple_of(r * _K, 8), _K)
            pltpu.sync_copy(obneg, os_hbm.at[ro])
            pltpu.sync_copy(obzero, oy1_hbm.at[ro])
            pltpu.sync_copy(obzero, ox1_hbm.at[ro])
            pltpu.sync_copy(obzero, oy2_hbm.at[ro])
            pltpu.sync_copy(obzero, ox2_hbm.at[ro])
            pltpu.sync_copy(sv, os_hbm.at[dv])
            pltpu.sync_copy(y1v, oy1_hbm.at[dv])
            pltpu.sync_copy(x1v, ox1_hbm.at[dv])
            pltpu.sync_copy(y2v, oy2_hbm.at[dv])
            pltpu.sync_copy(x2v, ox2_hbm.at[dv])

    f = functools.partial(
        pl.kernel, mesh=mesh,
        out_type=[jax.ShapeDtypeStruct((R * _K,), jnp.float32)] * 5,
        scratch_types=[pltpu.VMEM((N,), jnp.float32),
                       pltpu.VMEM((N,), jnp.int32)]
        + [pltpu.VMEM((N,), jnp.float32)] * 4
        + [pltpu.VMEM((_K,), jnp.float32)] * 2,
    )(body)
    outs = f(s0.reshape(R * N), gdest.reshape(R * N), bxt.reshape(B * 4 * N))
    return tuple(o.reshape(R, _K) for o in outs)


def _nms_body(sc_ref, y1_ref, x1_ref, y2_ref, x2_ref, out_ref, outc_ref):
    s0 = sc_ref[0]
    y1 = y1_ref[0]
    x1 = x1_ref[0]
    y2 = y2_ref[0]
    x2 = x2_ref[0]
    C, K = s0.shape
    a2 = jnp.maximum(y2 - y1, 0.0) * jnp.maximum(x2 - x1, 0.0)

    iota_l = lax.broadcasted_iota(jnp.int32, (C, K), 1)
    col = lax.broadcasted_iota(jnp.int32, (C, 128), 1)
    zc = jnp.zeros((C, 128), jnp.float32)

    def nms_step(i, carry):
        s, os_, oy1, ox1, oy2, ox2 = carry
        mx = jnp.max(s, axis=1, keepdims=True)
        idx = jnp.min(jnp.where(s == mx, iota_l, _BIG), axis=1, keepdims=True)
        oh = iota_l == idx
        cy1 = jnp.sum(jnp.where(oh, y1, 0.0), axis=1, keepdims=True)
        cx1 = jnp.sum(jnp.where(oh, x1, 0.0), axis=1, keepdims=True)
        cy2 = jnp.sum(jnp.where(oh, y2, 0.0), axis=1, keepdims=True)
        cx2 = jnp.sum(jnp.where(oh, x2, 0.0), axis=1, keepdims=True)
        valid = mx > -1.0
        yy1 = jnp.maximum(cy1, y1)
        xx1 = jnp.maximum(cx1, x1)
        yy2 = jnp.minimum(cy2, y2)
        xx2 = jnp.minimum(cx2, x2)
        inter = jnp.maximum(yy2 - yy1, 0.0) * jnp.maximum(xx2 - xx1, 0.0)
        a1 = jnp.maximum(cy2 - cy1, 0.0) * jnp.maximum(cx2 - cx1, 0.0)
        union = a1 + a2 - inter
        iou = inter / jnp.maximum(union, 1e-8)
        supp = (iou > _IOU_THR) | oh
        s_next = jnp.where(valid & supp, _NEG, s)
        here = col == i
        os_ = jnp.where(here, jnp.where(valid, mx, -1.0), os_)
        oy1 = jnp.where(here, jnp.where(valid, cy1, 0.0), oy1)
        ox1 = jnp.where(here, jnp.where(valid, cx1, 0.0), ox1)
        oy2 = jnp.where(here, jnp.where(valid, cy2, 0.0), oy2)
        ox2 = jnp.where(here, jnp.where(valid, cx2, 0.0), ox2)
        return s_next, os_, oy1, ox1, oy2, ox2

    _, os_, oy1, ox1, oy2, ox2 = lax.fori_loop(
        0, _MAX_OUT, nms_step, (s0, zc, zc, zc, zc, zc))

    cls_i = lax.broadcasted_iota(jnp.int32, (C, 128), 0)
    in_range = col < _MAX_OUT
    fiota = jnp.where(in_range, cls_i * _MAX_OUT + col, _BIG)
    ssm0 = jnp.where(in_range, os_, _NEG)
    l_iota = lax.broadcasted_iota(jnp.int32, (1, 128), 1)
    z1 = jnp.zeros((1, 128), jnp.float32)
    zi = jnp.zeros((1, 128), jnp.int32)

    def merge_step(i, carry):
        ssm, vcnt, ms, mb1, mb2, mb3, mb4, mc = carry
        mx = jnp.max(ssm)
        fidx = jnp.min(jnp.where(ssm == mx, fiota, _BIG))
        oh = fiota == fidx
        here = l_iota == i
        ms = jnp.where(here, mx, ms)
        mb1 = jnp.where(here, jnp.sum(jnp.where(oh, oy1, 0.0)), mb1)
        mb2 = jnp.where(here, jnp.sum(jnp.where(oh, ox1, 0.0)), mb2)
        mb3 = jnp.where(here, jnp.sum(jnp.where(oh, oy2, 0.0)), mb3)
        mb4 = jnp.where(here, jnp.sum(jnp.where(oh, ox2, 0.0)), mb4)
        mc = jnp.where(here, fidx // _MAX_OUT, mc)
        vcnt = vcnt + (mx > -1.0).astype(jnp.int32)
        return jnp.where(oh, _NEG, ssm), vcnt, ms, mb1, mb2, mb3, mb4, mc

    _, vcnt, ms, mb1, mb2, mb3, mb4, mc = lax.fori_loop(
        0, _MAX_OUT, merge_step,
        (ssm0, jnp.int32(0), z1, z1, z1, z1, z1, zi))

    out_ref[0] = jnp.concatenate([ms, mb1, mb2, mb3, mb4], axis=0)
    outc_ref[0] = jnp.where(l_iota == _MAX_OUT, vcnt, mc)


def kernel(boxes, scores):
    B, N, _, _ = boxes.shape
    C = scores.shape[-1]
    sc_t = jnp.transpose(scores, (0, 2, 1))
    bx_t = jnp.transpose(boxes[:, :, 0, :], (0, 2, 1))

    s0, gdest = pl.pallas_call(
        _select_body,
        grid=(B,),
        in_specs=[pl.BlockSpec((1, C, N), lambda b: (b, 0, 0))],
        out_specs=[pl.BlockSpec((1, C, N), lambda b: (b, 0, 0))] * 2,
        out_shape=[
            jax.ShapeDtypeStruct((B, C, N), jnp.float32),
            jax.ShapeDtypeStruct((B, C, N), jnp.int32),
        ],
    )(sc_t)

    cs, cy1, cx1, cy2, cx2 = _compact_sc(
        s0.reshape(B * C, N), gdest.reshape(B * C, N), bx_t)

    out, outc = pl.pallas_call(
        _nms_body,
        grid=(B,),
        in_specs=[pl.BlockSpec((1, C, _K), lambda b: (b, 0, 0))] * 5,
        out_specs=[
            pl.BlockSpec((1, 5, 128), lambda b: (b, 0, 0)),
            pl.BlockSpec((1, 1, 128), lambda b: (b, 0, 0)),
        ],
        out_shape=[
            jax.ShapeDtypeStruct((B, 5, 128), jnp.float32),
            jax.ShapeDtypeStruct((B, 1, 128), jnp.int32),
        ],
    )(cs.reshape(B, C, _K), cy1.reshape(B, C, _K),
      cx1.reshape(B, C, _K), cy2.reshape(B, C, _K), cx2.reshape(B, C, _K))

    final_s = out[:, 0, :_MAX_OUT]
    final_b = jnp.stack(
        [out[:, 1, :_MAX_OUT], out[:, 2, :_MAX_OUT],
         out[:, 3, :_MAX_OUT], out[:, 4, :_MAX_OUT]], axis=-1)
    final_c = outc[:, 0, :_MAX_OUT]
    valid = outc[:, 0, _MAX_OUT]
    return final_b, final_s, final_c, valid


# speedup vs baseline: 83.0534x; 83.0534x over previous
import jax
import jax.numpy as jnp
from jax import lax
from jax.experimental import pallas as pl

_MAX_OUT = 100
_IOU_THR = 0.5
_SCORE_THR = 0.05
_PRE_NMS = 1000
_NEG = -1e9
_BIG = 2**30
_K = 1024


def _nms_body(sc_ref, bx_ref, out_ref, outc_ref):
    sc = sc_ref[0]
    bx = bx_ref[0]
    C, N = sc.shape
    y1 = bx[0:1, :]
    x1 = bx[1:2, :]
    y2 = bx[2:3, :]
    x2 = bx[3:4, :]

    bits = lax.bitcast_convert_type(sc, jnp.int32)
    lo = jnp.zeros((C, 1), jnp.int32)
    hi = jnp.max(bits, axis=1, keepdims=True) + 1

    def bs_step(_, carry):
        lo, hi = carry
        mid = (lo + hi) // 2
        cnt = jnp.sum((bits >= mid).astype(jnp.int32), axis=1, keepdims=True)
        ge = cnt >= _PRE_NMS
        return jnp.where(ge, mid, lo), jnp.where(ge, hi, mid)

    lo, hi = lax.fori_loop(0, 31, bs_step, (lo, hi))
    vstar = lo
    gt = bits > vstar
    eq = bits == vstar
    n_gt = jnp.sum(gt.astype(jnp.int32), axis=1, keepdims=True)
    m = _PRE_NMS - n_gt
    eqrank = eq.astype(jnp.int32)
    shift = 1
    while shift < N:
        z = jnp.zeros((C, shift), jnp.int32)
        eqrank = eqrank + jnp.concatenate([z, eqrank[:, :N - shift]], axis=1)
        shift *= 2
    sel = gt | (eq & (eqrank <= m))
    m2 = sel & (sc > _SCORE_THR)

    iota_n = lax.broadcasted_iota(jnp.int32, (C, N), 1)
    rank = m2.astype(jnp.int32)
    shift = 1
    while shift < N:
        z = jnp.zeros((C, shift), jnp.int32)
        rank = rank + jnp.concatenate([z, rank[:, :N - shift]], axis=1)
        shift *= 2
    d = iota_n - (rank - 1)
    sval = jnp.where(m2, sc, -1.0)
    y1b = jnp.broadcast_to(y1, (C, N))
    x1b = jnp.broadcast_to(x1, (C, N))
    y2b = jnp.broadcast_to(y2, (C, N))
    x2b = jnp.broadcast_to(x2, (C, N))
    flag = m2
    k = 0
    while (1 << k) < N:
        s = 1 << k

        def shl(x, s=s):
            return jnp.concatenate([x[:, s:], x[:, :s]], axis=1)

        mv = flag & (((d >> k) & 1) == 1)
        mv_in = shl(mv.astype(jnp.int32)) == 1
        sval = jnp.where(mv_in, shl(sval), sval)
        y1b = jnp.where(mv_in, shl(y1b), y1b)
        x1b = jnp.where(mv_in, shl(x1b), x1b)
        y2b = jnp.where(mv_in, shl(y2b), y2b)
        x2b = jnp.where(mv_in, shl(x2b), x2b)
        d = jnp.where(mv_in, shl(d), d)
        flag = mv_in | (flag & ~mv)
        k += 1

    fl = flag[:, :_K]
    s0 = jnp.where(fl, sval[:, :_K], -1.0)
    y1 = jnp.where(fl, y1b[:, :_K], 0.0)
    x1 = jnp.where(fl, x1b[:, :_K], 0.0)
    y2 = jnp.where(fl, y2b[:, :_K], 0.0)
    x2 = jnp.where(fl, x2b[:, :_K], 0.0)
    a2 = jnp.maximum(y2 - y1, 0.0) * jnp.maximum(x2 - x1, 0.0)

    iota_l = lax.broadcasted_iota(jnp.int32, (C, _K), 1)
    col = lax.broadcasted_iota(jnp.int32, (C, 128), 1)
    zc = jnp.zeros((C, 128), jnp.float32)

    def nms_step(i, carry):
        s, os_, oy1, ox1, oy2, ox2 = carry
        mx = jnp.max(s, axis=1, keepdims=True)
        idx = jnp.min(jnp.where(s == mx, iota_l, _BIG), axis=1, keepdims=True)
        oh = iota_l == idx
        cy1 = jnp.sum(jnp.where(oh, y1, 0.0), axis=1, keepdims=True)
        cx1 = jnp.sum(jnp.where(oh, x1, 0.0), axis=1, keepdims=True)
        cy2 = jnp.sum(jnp.where(oh, y2, 0.0), axis=1, keepdims=True)
        cx2 = jnp.sum(jnp.where(oh, x2, 0.0), axis=1, keepdims=True)
        valid = mx > -1.0
        yy1 = jnp.maximum(cy1, y1)
        xx1 = jnp.maximum(cx1, x1)
        yy2 = jnp.minimum(cy2, y2)
        xx2 = jnp.minimum(cx2, x2)
        inter = jnp.maximum(yy2 - yy1, 0.0) * jnp.maximum(xx2 - xx1, 0.0)
        a1 = jnp.maximum(cy2 - cy1, 0.0) * jnp.maximum(cx2 - cx1, 0.0)
        union = a1 + a2 - inter
        iou = inter / jnp.maximum(union, 1e-8)
        supp = (iou > _IOU_THR) | oh
        s_next = jnp.where(valid & supp, _NEG, s)
        here = col == i
        os_ = jnp.where(here, jnp.where(valid, mx, -1.0), os_)
        oy1 = jnp.where(here, jnp.where(valid, cy1, 0.0), oy1)
        ox1 = jnp.where(here, jnp.where(valid, cx1, 0.0), ox1)
        oy2 = jnp.where(here, jnp.where(valid, cy2, 0.0), oy2)
        ox2 = jnp.where(here, jnp.where(valid, cx2, 0.0), ox2)
        return s_next, os_, oy1, ox1, oy2, ox2

    _, os_, oy1, ox1, oy2, ox2 = lax.fori_loop(
        0, _MAX_OUT, nms_step, (s0, zc, zc, zc, zc, zc))

    cls_i = lax.broadcasted_iota(jnp.int32, (C, 128), 0)
    in_range = col < _MAX_OUT
    fiota = jnp.where(in_range, cls_i * _MAX_OUT + col, _BIG)
    ssm0 = jnp.where(in_range, os_, _NEG)
    l_iota = lax.broadcasted_iota(jnp.int32, (1, 128), 1)
    z1 = jnp.zeros((1, 128), jnp.float32)
    zi = jnp.zeros((1, 128), jnp.int32)

    def merge_step(i, carry):
        ssm, vcnt, ms, mb1, mb2, mb3, mb4, mc = carry
        mx = jnp.max(ssm)
        fidx = jnp.min(jnp.where(ssm == mx, fiota, _BIG))
        oh = fiota == fidx
        here = l_iota == i
        ms = jnp.where(here, mx, ms)
        mb1 = jnp.where(here, jnp.sum(jnp.where(oh, oy1, 0.0)), mb1)
        mb2 = jnp.where(here, jnp.sum(jnp.where(oh, ox1, 0.0)), mb2)
        mb3 = jnp.where(here, jnp.sum(jnp.where(oh, oy2, 0.0)), mb3)
        mb4 = jnp.where(here, jnp.sum(jnp.where(oh, ox2, 0.0)), mb4)
        mc = jnp.where(here, fidx // _MAX_OUT, mc)
        vcnt = vcnt + (mx > -1.0).astype(jnp.int32)
        return jnp.where(oh, _NEG, ssm), vcnt, ms, mb1, mb2, mb3, mb4, mc

    _, vcnt, ms, mb1, mb2, mb3, mb4, mc = lax.fori_loop(
        0, _MAX_OUT, merge_step,
        (ssm0, jnp.int32(0), z1, z1, z1, z1, z1, zi))

    out_ref[0] = jnp.concatenate([ms, mb1, mb2, mb3, mb4], axis=0)
    outc_ref[0] = jnp.where(l_iota == _MAX_OUT, vcnt, mc)


def kernel(boxes, scores):
    B, N, _, _ = boxes.shape
    C = scores.shape[-1]
    sc_t = jnp.transpose(scores, (0, 2, 1))
    bx_t = jnp.transpose(boxes[:, :, 0, :], (0, 2, 1))
    out, outc = pl.pallas_call(
        _nms_body,
        grid=(B,),
        in_specs=[
            pl.BlockSpec((1, C, N), lambda b: (b, 0, 0)),
            pl.BlockSpec((1, 4, N), lambda b: (b, 0, 0)),
        ],
        out_specs=[
            pl.BlockSpec((1, 5, 128), lambda b: (b, 0, 0)),
            pl.BlockSpec((1, 1, 128), lambda b: (b, 0, 0)),
        ],
        out_shape=[
            jax.ShapeDtypeStruct((B, 5, 128), jnp.float32),
            jax.ShapeDtypeStruct((B, 1, 128), jnp.int32),
        ],
    )(sc_t, bx_t)
    final_s = out[:, 0, :_MAX_OUT]
    final_b = jnp.stack(
        [out[:, 1, :_MAX_OUT], out[:, 2, :_MAX_OUT],
         out[:, 3, :_MAX_OUT], out[:, 4, :_MAX_OUT]], axis=-1)
    final_c = outc[:, 0, :_MAX_OUT]
    valid = outc[:, 0, _MAX_OUT]
    return final_b, final_s, final_c, valid
